# baseline (speedup 1.0000x reference)
import jax
import jax.numpy as jnp
from jax import lax
from jax.experimental import pallas as pl
from jax.experimental.pallas import tpu as pltpu
from jax.experimental.pallas import tpu_sc as plsc

N = 10000
E = 320000
EP = 327680
D = 128
R = 16
H = 128
CUTOFF = 5.0
NP = 10240
NHI = NP // 128
BE = 4096
BN = 2000
BND = 2560
CHUNK = 64
NCT = EP // CHUNK // 32
NTILES = 32


def _dot_t(a, w):
    return lax.dot_general(a, w, (((1,), (1,)), ((), ())),
                           preferred_element_type=jnp.float32)


_LOG2E = 1.4426950408889634


def _exp2(t):
    t = jnp.clip(t, -90.0, 90.0)
    n = jnp.floor(t)
    f = t - n
    p = 7.9019940e-2
    p = p * f + 2.2412644e-1
    p = p * f + 6.9683858e-1
    p = p * f + 9.9981196e-1
    ni = n.astype(jnp.int32)
    scale = lax.bitcast_convert_type(
        lax.shift_left(ni + 127, 23), jnp.float32)
    return p * scale


def _recip_pos(d):
    r = lax.bitcast_convert_type(
        jnp.int32(0x7EF311C3) - lax.bitcast_convert_type(d, jnp.int32),
        jnp.float32)
    r = r * (2.0 - d * r)
    r = r * (2.0 - d * r)
    return r


def _sigmoid(z):
    return _recip_pos(1.0 + _exp2(-_LOG2E * z))


def _silu(z):
    return z * _sigmoid(z)


def _edge_body(feat_ref, wm1, wmb1, wm2, wmb2, wm3, wmb3,
               eg1, egb1, eg2, egb2, rwg_ref, nm_ref):
    feat = feat_ref[...]
    r = feat[:, 0:1]
    dvf = feat[:, 1:2]
    g = _silu(_dot_t(feat, eg1[...]) + egb1[...])
    gate = _sigmoid(
        jnp.sum(g * eg2[...], axis=1, keepdims=True) + egb2[...])
    cc = 0.5 * (jnp.cos((jnp.pi / CUTOFF) * r) + 1.0)
    cc = cc * (r <= CUTOFF).astype(jnp.float32)
    ew = cc * gate
    h = _silu(_dot_t(feat, wm1[...]) + wmb1[...])
    h = _silu(_dot_t(h, wm2[...]) + wmb2[...])
    rw = _dot_t(h, wm3[...]) + wmb3[...]

    def _bcast(col, k):
        return lax.dot_general(col, jnp.ones((1, k), jnp.float32),
                               (((1,), (0,)), ((), ())),
                               preferred_element_type=jnp.float32)

    rwg_ref[...] = rw * _bcast(ew, D)

    hi = jnp.floor(dvf * (1.0 / 128.0))
    lo = dvf - 128.0 * hi
    ia = lax.broadcasted_iota(jnp.int32, (feat.shape[0], NHI), 1).astype(
        jnp.float32)
    ib = lax.broadcasted_iota(jnp.int32, (feat.shape[0], 128), 1).astype(
        jnp.float32)
    a = jnp.where(ia == _bcast(hi, NHI), _bcast(ew, NHI), 0.0)
    b = jnp.where(ib == _bcast(lo, 128), 1.0, 0.0)
    contrib = lax.dot_general(a, b, (((0,), (0,)), ((), ())),
                              preferred_element_type=jnp.float32)

    @pl.when(pl.program_id(0) == 0)
    def _init():
        nm_ref[...] = jnp.zeros_like(nm_ref)

    nm_ref[...] += contrib


def _ln_body(x_ref, w_ref, b_ref, out_ref):
    xv = x_ref[...]
    mu = jnp.mean(xv, axis=1, keepdims=True)
    var = jnp.mean((xv - mu) ** 2, axis=1, keepdims=True)
    out_ref[...] = (xv - mu) * lax.rsqrt(var + 1e-5) * w_ref[...] + b_ref[...]


def _node_body(mp_ref, nm_ref, xn_ref, x_ref, mm1, mmb1, mm2, mmb2,
               slw, slb, ulw, ulb, rs_ref, out_ref):
    tot = mp_ref[0] + mp_ref[1]
    agg = tot / jnp.maximum(nm_ref[...], 1e-8)
    h = _silu(_dot_t(agg, mm1[...]) + mmb1[...])
    ao = _dot_t(h, mm2[...]) + mmb2[...]
    xn = xn_ref[...]
    out = _dot_t(xn, slw[...]) + slb[...] + _dot_t(ao, ulw[...]) + ulb[...]
    out_ref[...] = x_ref[...] + rs_ref[0, 0] * out


def _sc_agg_body(xn_hbm, rwg_hbm, src_hbm, dst_hbm, mparts_hbm,
                 srcb, dstb, dsts, gath, rwgb, acc,
                 gs0, gs1, rs0, rs1, ss0, ss1):
    c = lax.axis_index("c")
    s = lax.axis_index("s")
    wid = s * 2 + c
    gsems = (gs0, gs1)
    rsems = (rs0, rs1)
    ssems = (ss0, ss1)
    half = NCT // 2

    z16 = jnp.zeros((16,), jnp.float32)

    def zrow(i, carry):
        for k in range(D // 16):
            gath[0, i, pl.ds(k * 16, 16)] = z16
        return carry

    lax.fori_loop(0, CHUNK, zrow, 0)
    for t in range(10):
        pltpu.sync_copy(gath.at[0], acc.at[pl.ds(s * 640 + t * CHUNK, CHUNK)])
    plsc.subcore_barrier()

    pltpu.sync_copy(dst_hbm.at[pl.ds(wid * (NCT // 2), NCT // 2)], dstb)

    def wait_g(b):
        pltpu.make_async_copy(xn_hbm.at[pl.ds(0, CHUNK)], gath.at[b],
                              gsems[b]).wait()

    def wait_r(b):
        pltpu.make_async_copy(rwg_hbm.at[pl.ds(0, CHUNK)], rwgb.at[b],
                              rsems[b]).wait()

    def wait_s(b):
        pltpu.make_async_copy(gath.at[b], acc.at[dsts.at[0]], ssems[b]).wait()

    for h in (0, 1):
        base0 = (wid * NCT + h * half) * CHUNK
        pltpu.sync_copy(
            src_hbm.at[pl.ds(wid * (NCT // 2) + h * (half // 2), half // 2)],
            srcb)

        def issue(row, col, b):
            base = base0 + (2 * row + col // CHUNK) * CHUNK
            pltpu.async_copy(xn_hbm.at[srcb.at[row, pl.ds(col, CHUNK)]],
                             gath.at[b], gsems[b])
            pltpu.async_copy(rwg_hbm.at[pl.ds(base, CHUNK)], rwgb.at[b],
                             rsems[b])

        issue(0, 0, 0)

        def pair(jj, carry):
            for b in (0, 1):
                lj = 2 * jj + b
                nb = 1 - b

                @pl.when(lj + 1 < half)
                def _pref():
                    @pl.when(lj >= 1)
                    def _drain():
                        wait_s(nb)

                    if b == 0:
                        issue(jj, CHUNK, nb)
                    else:
                        issue(jj + 1, 0, nb)

                wait_g(b)
                wait_r(b)

                def mrow(i, cc2):
                    for k in range(D // 16):
                        sl = pl.ds(k * 16, 16)
                        gath[b, i, sl] = gath[b, i, sl] * rwgb[b, i, sl]
                    return cc2

                lax.fori_loop(0, CHUNK, mrow, 0)
                row_d = h * (half // 2) + jj
                for k in range(CHUNK // 16):
                    dsts[b, pl.ds(k * 16, 16)] = (
                        dstb[row_d, pl.ds(b * CHUNK + k * 16, 16)])
                pltpu.async_copy(gath.at[b], acc.at[dsts.at[b]],
                                 ssems[b], add=True)
            return carry

        lax.fori_loop(0, half // 2, pair, 0)
        wait_s(0)
        wait_s(1)

    plsc.subcore_barrier()

    for t in range(10):
        row = s * 640 + t * CHUNK
        pltpu.sync_copy(acc.at[pl.ds(row, CHUNK)],
                        mparts_hbm.at[c, pl.ds(row, CHUNK)])


def _full(shape):
    zeros = (0,) * len(shape)
    return pl.BlockSpec(shape, lambda i, z=zeros: z)


def kernel(x, edge_src, edge_dst, edge_sh, edge_rbf, edge_len,
           ln_w, ln_b, wm_w1, wm_b1, wm_w2, wm_b2, wm_w3, wm_b3,
           eg_w1, eg_b1, eg_w2, eg_b2, mm_w1, mm_b1, mm_w2, mm_b2,
           sl_w, sl_b, ul_w, ul_b, res_scale):
    del edge_sh
    f32 = jnp.float32

    pad = EP - E
    len_p = jnp.concatenate([edge_len, jnp.full((pad,), 2.0 * CUTOFF, f32)])
    dst_p = jnp.concatenate([edge_dst, jnp.zeros((pad,), edge_dst.dtype)])
    src_p = jnp.concatenate([edge_src, jnp.zeros((pad,), edge_src.dtype)])
    rbf_p = jnp.pad(edge_rbf, ((0, pad), (0, 0)))

    rwg, nmat = pl.pallas_call(
        _edge_body,
        grid=(EP // BE,),
        in_specs=[
            pl.BlockSpec((BE, R + 2), lambda i: (i, 0)),
            _full((H, R + 2)), _full((1, H)),
            _full((H, H)), _full((1, H)),
            _full((D, H)), _full((1, D)),
            _full((H, R + 2)), _full((1, H)),
            _full((1, H)), _full((1, 1)),
        ],
        out_specs=[
            pl.BlockSpec((BE, D), lambda i: (i, 0)),
            pl.BlockSpec((NHI, 128), lambda i: (0, 0)),
        ],
        out_shape=[
            jax.ShapeDtypeStruct((EP, D), f32),
            jax.ShapeDtypeStruct((NHI, 128), f32),
        ],
    )(jnp.concatenate(
        [len_p[:, None], dst_p.astype(f32)[:, None], rbf_p], axis=1),
      jnp.pad(wm_w1, ((0, 0), (2, 0))), wm_b1.reshape(1, H),
      wm_w2, wm_b2.reshape(1, H),
      wm_w3, wm_b3.reshape(1, D),
      jnp.pad(eg_w1, ((0, 0), (2, 0))), eg_b1.reshape(1, H),
      eg_w2, eg_b2.reshape(1, 1))

    xn = pl.pallas_call(
        _ln_body,
        grid=(N // BN,),
        in_specs=[
            pl.BlockSpec((BN, D), lambda i: (i, 0)),
            _full((1, D)), _full((1, D)),
        ],
        out_specs=pl.BlockSpec((BN, D), lambda i: (i, 0)),
        out_shape=jax.ShapeDtypeStruct((N, D), f32),
    )(x, ln_w.reshape(1, D), ln_b.reshape(1, D))

    mesh = plsc.VectorSubcoreMesh(core_axis_name="c", subcore_axis_name="s")
    mparts = pl.kernel(
        _sc_agg_body,
        out_type=jax.ShapeDtypeStruct((2, NP, D), f32),
        mesh=mesh,
        compiler_params=pltpu.CompilerParams(needs_layout_passes=False),
        scratch_types=[
            pltpu.VMEM((NCT // 4, 2 * CHUNK), jnp.int32),
            pltpu.VMEM((NCT // 2, 2 * CHUNK), jnp.int32),
            pltpu.VMEM((2, CHUNK), jnp.int32),
            pltpu.VMEM((2, CHUNK, D), f32),
            pltpu.VMEM((2, CHUNK, D), f32),
            pltpu.VMEM_SHARED((NP, D), f32),
            pltpu.SemaphoreType.DMA,
            pltpu.SemaphoreType.DMA,
            pltpu.SemaphoreType.DMA,
            pltpu.SemaphoreType.DMA,
            pltpu.SemaphoreType.DMA,
            pltpu.SemaphoreType.DMA,
        ],
    )(xn, rwg, src_p.reshape(EP // (2 * CHUNK), 2 * CHUNK),
      dst_p.reshape(EP // (2 * CHUNK), 2 * CHUNK))

    out = pl.pallas_call(
        _node_body,
        grid=(pl.cdiv(N, BND),),
        in_specs=[
            pl.BlockSpec((2, BND, D), lambda i: (0, i, 0)),
            pl.BlockSpec((BND, 1), lambda i: (i, 0)),
            pl.BlockSpec((BND, D), lambda i: (i, 0)),
            pl.BlockSpec((BND, D), lambda i: (i, 0)),
            _full((H, D)), _full((1, H)),
            _full((D, H)), _full((1, D)),
            _full((D, D)), _full((1, D)),
            _full((D, D)), _full((1, D)),
            pl.BlockSpec((1, 1), lambda i: (0, 0), memory_space=pltpu.SMEM),
        ],
        out_specs=pl.BlockSpec((BND, D), lambda i: (i, 0)),
        out_shape=jax.ShapeDtypeStruct((N, D), f32),
    )(mparts, nmat.reshape(NP, 1), xn, x,
      mm_w1, mm_b1.reshape(1, H), mm_w2, mm_b2.reshape(1, D),
      sl_w, sl_b.reshape(1, D), ul_w, ul_b.reshape(1, D),
      res_scale.reshape(1, 1))
    return out
